# Initial kernel scaffold; baseline (speedup 1.0000x reference)
import jax
import jax.numpy as jnp
from jax.experimental import pallas as pl


def kernel(x, edge_index, W, b):
    raise NotImplementedError("write your pallas kernel here")


# speedup vs baseline: 30.7872x; 30.7872x over previous
import functools

import jax
import jax.numpy as jnp
from jax import lax
from jax.experimental import pallas as pl
from jax.experimental.pallas import tpu as pltpu
from jax.experimental.pallas import tpu_sc as plsc

NC = 2
NS = 16
NW = NC * NS
L = 16

EC = 125
N_PAD = 10240


def _hist_body(nchunk, row_hbm, cnt_hbm, row_v, ones_v, z_v, hist_sh, sem):
    del sem
    cid = lax.axis_index("c")
    sid = lax.axis_index("s")
    wid = cid * NS + sid
    rows_per_tile = N_PAD // NS

    pltpu.sync_copy(row_hbm.at[wid], row_v)

    def fill_ones(i, c):
        ones_v[pl.ds(i * L, L)] = jnp.full((L,), 1.0, jnp.float32)
        return c

    lax.fori_loop(0, EC // L + 1, fill_ones, 0)

    def fill_zeros(i, c):
        z_v[pl.ds(i * L, L)] = jnp.zeros((L,), jnp.float32)
        return c

    lax.fori_loop(0, rows_per_tile // L, fill_zeros, 0)
    pltpu.sync_copy(z_v, hist_sh.at[pl.ds(sid * rows_per_tile, rows_per_tile)])
    plsc.subcore_barrier()

    def step(j, c):
        pltpu.sync_copy(ones_v.at[pl.ds(0, EC)], hist_sh.at[row_v.at[j]], add=True)
        return c

    lax.fori_loop(0, nchunk, step, 0)
    plsc.subcore_barrier()
    pltpu.sync_copy(
        hist_sh.at[pl.ds(sid * rows_per_tile, rows_per_tile)],
        cnt_hbm.at[cid].at[pl.ds(sid * rows_per_tile, rows_per_tile)],
    )


def _degree_histogram(row_r):
    nchunk = row_r.shape[1]
    mesh = plsc.VectorSubcoreMesh(core_axis_name="c", subcore_axis_name="s")
    return pl.kernel(
        functools.partial(_hist_body, nchunk),
        out_type=jax.ShapeDtypeStruct((NC, N_PAD), jnp.float32),
        mesh=mesh,
        scratch_types=[
            pltpu.VMEM(row_r.shape[1:], jnp.int32),
            pltpu.VMEM(((EC // L + 1) * L,), jnp.float32),
            pltpu.VMEM((N_PAD // NS,), jnp.float32),
            pltpu.VMEM_SHARED((N_PAD,), jnp.float32),
            pltpu.SemaphoreType.DMA,
        ],
    )(row_r)


def _prep_body(x_ref, w_ref, cnt_ref, hs_ref, dis_ref):
    cnt = cnt_ref[...]
    deg = cnt[:, 0:1] + cnt[:, 1:2] + 1.0
    dis = lax.rsqrt(deg)
    h = jnp.dot(x_ref[...], w_ref[...], preferred_element_type=jnp.float32)
    hs_ref[...] = h * dis
    dis_ref[...] = dis


def _prep(x, W, cnt_t):
    n, d = x.shape
    blk = 1000
    grid = n // blk
    return pl.pallas_call(
        _prep_body,
        grid=(grid,),
        in_specs=[
            pl.BlockSpec((blk, d), lambda i: (i, 0)),
            pl.BlockSpec((d, d), lambda i: (0, 0)),
            pl.BlockSpec((blk, 2), lambda i: (i, 0)),
        ],
        out_specs=[
            pl.BlockSpec((blk, d), lambda i: (i, 0)),
            pl.BlockSpec((blk, 1), lambda i: (i, 0)),
        ],
        out_shape=[
            jax.ShapeDtypeStruct((n, d), jnp.float32),
            jax.ShapeDtypeStruct((n, 1), jnp.float32),
        ],
    )(x, W, cnt_t)


def _scatter_body(nchunk, hs_hbm, row_hbm, col_hbm, part_hbm,
                  row_v, col_v, gbuf, acc_sh, sem):
    del sem
    cid = lax.axis_index("c")
    sid = lax.axis_index("s")
    wid = cid * NS + sid
    rows_per_tile = N_PAD // NS
    gb = gbuf.shape[0]

    pltpu.sync_copy(row_hbm.at[wid], row_v)
    pltpu.sync_copy(col_hbm.at[wid], col_v)

    def zero_row(i, c):
        for k in range(gbuf.shape[1] // L):
            gbuf[i, pl.ds(k * L, L)] = jnp.zeros((L,), jnp.float32)
        return c

    lax.fori_loop(0, gb, zero_row, 0)
    for k in range(rows_per_tile // gb):
        pltpu.sync_copy(gbuf, acc_sh.at[pl.ds(sid * rows_per_tile + k * gb, gb)])
    plsc.subcore_barrier()

    def step(j, c):
        pltpu.sync_copy(hs_hbm.at[row_v.at[j]], gbuf.at[pl.ds(0, EC)])
        pltpu.sync_copy(gbuf.at[pl.ds(0, EC)], acc_sh.at[col_v.at[j]], add=True)
        return c

    lax.fori_loop(0, nchunk, step, 0)
    plsc.subcore_barrier()
    pltpu.sync_copy(
        acc_sh.at[pl.ds(sid * rows_per_tile, rows_per_tile)],
        part_hbm.at[cid].at[pl.ds(sid * rows_per_tile, rows_per_tile)],
    )


def _edge_scatter(hs, row_r, col_r):
    d = hs.shape[1]
    nchunk = row_r.shape[1]
    mesh = plsc.VectorSubcoreMesh(core_axis_name="c", subcore_axis_name="s")
    return pl.kernel(
        functools.partial(_scatter_body, nchunk),
        out_type=jax.ShapeDtypeStruct((NC, N_PAD, d), jnp.float32),
        mesh=mesh,
        scratch_types=[
            pltpu.VMEM(row_r.shape[1:], jnp.int32),
            pltpu.VMEM(col_r.shape[1:], jnp.int32),
            pltpu.VMEM((128, d), jnp.float32),
            pltpu.VMEM_SHARED((N_PAD, d), jnp.float32),
            pltpu.SemaphoreType.DMA,
        ],
    )(hs, row_r, col_r)


def _final_body(p0_ref, p1_ref, hs_ref, dis_ref, b_ref, o_ref):
    o_ref[...] = (p0_ref[...] + p1_ref[...] + hs_ref[...]) * dis_ref[...] + b_ref[...]


def _final(p0, p1, hs, dis, b2):
    n, d = hs.shape
    blk = 1000
    grid = n // blk
    row_spec = pl.BlockSpec((blk, d), lambda i: (i, 0))
    return pl.pallas_call(
        _final_body,
        grid=(grid,),
        in_specs=[
            row_spec,
            row_spec,
            row_spec,
            pl.BlockSpec((blk, 1), lambda i: (i, 0)),
            pl.BlockSpec((1, d), lambda i: (0, 0)),
        ],
        out_specs=row_spec,
        out_shape=jax.ShapeDtypeStruct((n, d), jnp.float32),
    )(p0, p1, hs, dis, b2)


@jax.jit
def kernel(x, edge_index, W, b):
    n, d = x.shape
    e = edge_index.shape[1]
    assert e % (NW * EC) == 0 and n <= N_PAD

    nchunk = e // (NW * EC)
    row_r = edge_index[0].reshape(NW, nchunk, EC)
    col_r = edge_index[1].reshape(NW, nchunk, EC)

    cnt = _degree_histogram(row_r)
    cnt_t = cnt[:, :n].T
    hs, dis = _prep(x, W, cnt_t)
    parts = _edge_scatter(hs, row_r, col_r)
    return _final(parts[0, :n], parts[1, :n], hs, dis, b.reshape(1, d))


# speedup vs baseline: 39.3761x; 1.2790x over previous
import functools

import jax
import jax.numpy as jnp
from jax import lax
from jax.experimental import pallas as pl
from jax.experimental.pallas import tpu as pltpu
from jax.experimental.pallas import tpu_sc as plsc

NC = 2
NS = 16
NW = NC * NS
L = 16

ECH = 100
EC = 125
NB = 2
N_PAD = 10240


def _hist_body(nchunk, row_hbm, cnt_hbm, row_v, ones_v, z_v, hist_sh, sem):
    del sem
    cid = lax.axis_index("c")
    sid = lax.axis_index("s")
    wid = cid * NS + sid
    rows_per_tile = N_PAD // NS

    pltpu.sync_copy(row_hbm.at[wid], row_v)

    def fill_ones(i, c):
        ones_v[pl.ds(i * L, L)] = jnp.full((L,), 1.0, jnp.float32)
        return c

    lax.fori_loop(0, ECH // L + 1, fill_ones, 0)

    def fill_zeros(i, c):
        z_v[pl.ds(i * L, L)] = jnp.zeros((L,), jnp.float32)
        return c

    lax.fori_loop(0, rows_per_tile // L, fill_zeros, 0)
    pltpu.sync_copy(z_v, hist_sh.at[pl.ds(sid * rows_per_tile, rows_per_tile)])
    plsc.subcore_barrier()

    def step(j, c):
        pltpu.sync_copy(ones_v.at[pl.ds(0, ECH)], hist_sh.at[row_v.at[j]], add=True)
        return c

    lax.fori_loop(0, nchunk, step, 0)
    plsc.subcore_barrier()
    pltpu.sync_copy(
        hist_sh.at[pl.ds(sid * rows_per_tile, rows_per_tile)],
        cnt_hbm.at[cid].at[pl.ds(sid * rows_per_tile, rows_per_tile)],
    )


def _degree_histogram(row_r):
    nchunk = row_r.shape[1]
    mesh = plsc.VectorSubcoreMesh(core_axis_name="c", subcore_axis_name="s")
    return pl.kernel(
        functools.partial(_hist_body, nchunk),
        out_type=jax.ShapeDtypeStruct((NC, N_PAD), jnp.float32),
        mesh=mesh,
        scratch_types=[
            pltpu.VMEM(row_r.shape[1:], jnp.int32),
            pltpu.VMEM(((ECH // L + 1) * L,), jnp.float32),
            pltpu.VMEM((N_PAD // NS,), jnp.float32),
            pltpu.VMEM_SHARED((N_PAD,), jnp.float32),
            pltpu.SemaphoreType.DMA,
        ],
    )(row_r)


def _prep_body(x_ref, w_ref, cnt_ref, hs_ref, dis_ref):
    cnt = cnt_ref[...]
    deg = cnt[:, 0:1] + cnt[:, 1:2] + 1.0
    dis = lax.rsqrt(deg)
    h = jnp.dot(x_ref[...], w_ref[...], preferred_element_type=jnp.float32)
    hs_ref[...] = h * dis
    dis_ref[...] = dis


def _prep(x, W, cnt_t):
    n, d = x.shape
    blk = 1000
    grid = n // blk
    return pl.pallas_call(
        _prep_body,
        grid=(grid,),
        in_specs=[
            pl.BlockSpec((blk, d), lambda i: (i, 0)),
            pl.BlockSpec((d, d), lambda i: (0, 0)),
            pl.BlockSpec((blk, 2), lambda i: (i, 0)),
        ],
        out_specs=[
            pl.BlockSpec((blk, d), lambda i: (i, 0)),
            pl.BlockSpec((blk, 1), lambda i: (i, 0)),
        ],
        out_shape=[
            jax.ShapeDtypeStruct((n, d), jnp.float32),
            jax.ShapeDtypeStruct((n, 1), jnp.float32),
        ],
    )(x, W, cnt_t)


def _scatter_body(nchunk, hs_hbm, row_hbm, col_hbm, part_hbm,
                  ri, col_v, bufs, acc_sh, *sems):
    isem = sems[:NB]
    gsem = sems[NB:]
    cid = lax.axis_index("c")
    sid = lax.axis_index("s")
    wid = cid * NS + sid
    rows_per_tile = N_PAD // NS
    zb = 128

    rsrc = row_hbm.at[wid]
    pltpu.sync_copy(col_hbm.at[wid], col_v)

    def zero_row(i, c):
        for k in range(bufs.shape[1] // L):
            bufs[i, pl.ds(k * L, L)] = jnp.zeros((L,), jnp.float32)
        return c

    lax.fori_loop(0, zb, zero_row, 0)
    for k in range(rows_per_tile // zb):
        pltpu.sync_copy(bufs.at[pl.ds(0, zb)],
                        acc_sh.at[pl.ds(sid * rows_per_tile + k * zb, zb)])
    plsc.subcore_barrier()

    def buf(i):
        return bufs.at[pl.ds(i * EC, EC)]

    def idx_load(j, i):
        pltpu.async_copy(rsrc.at[j], ri.at[i], isem[i])

    def idx_wait(j, i):
        pltpu.make_async_copy(rsrc.at[j], ri.at[i], isem[i]).wait()

    def gather(j, i):
        del j
        pltpu.async_copy(hs_hbm.at[ri.at[i]], buf(i), gsem[i])

    def gather_wait(j, i):
        del j
        pltpu.make_async_copy(hs_hbm.at[ri.at[i]], buf(i), gsem[i]).wait()

    def scatter(j, i):
        pltpu.sync_copy(buf(i), acc_sh.at[col_v.at[j]], add=True)

    idx_load(0, 0)
    idx_load(1, 1)
    idx_wait(0, 0)
    gather(0, 0)

    def body(k, c):
        for i in range(NB):
            j = NB * k + i
            i2 = (i + 1) % NB
            idx_wait(j + 1, i2)
            gather(j + 1, i2)
            gather_wait(j, i)
            scatter(j, i)
            idx_load(j + 2, i)
        return c

    lax.fori_loop(0, (nchunk - NB) // NB, body, 0)
    j = nchunk - NB
    for i0 in range(NB):
        i = (j + i0) % NB
        if i0 == 0:
            i2 = (i + 1) % NB
            idx_wait(j + 1, i2)
            gather(j + 1, i2)
        gather_wait(j + i0, i)
        scatter(j + i0, i)

    plsc.subcore_barrier()
    pltpu.sync_copy(
        acc_sh.at[pl.ds(sid * rows_per_tile, rows_per_tile)],
        part_hbm.at[cid].at[pl.ds(sid * rows_per_tile, rows_per_tile)],
    )


def _edge_scatter(hs, row_r, col_r):
    d = hs.shape[1]
    nchunk = row_r.shape[1]
    mesh = plsc.VectorSubcoreMesh(core_axis_name="c", subcore_axis_name="s")
    return pl.kernel(
        functools.partial(_scatter_body, nchunk),
        out_type=jax.ShapeDtypeStruct((NC, N_PAD, d), jnp.float32),
        mesh=mesh,
        scratch_types=[
            pltpu.VMEM((NB, EC), jnp.int32),
            pltpu.VMEM(col_r.shape[1:], jnp.int32),
            pltpu.VMEM((NB * EC, d), jnp.float32),
            pltpu.VMEM_SHARED((N_PAD, d), jnp.float32),
        ] + [pltpu.SemaphoreType.DMA] * (2 * NB),
    )(hs, row_r, col_r)


def _final_body(p0_ref, p1_ref, hs_ref, dis_ref, b_ref, o_ref):
    o_ref[...] = (p0_ref[...] + p1_ref[...] + hs_ref[...]) * dis_ref[...] + b_ref[...]


def _final(p0, p1, hs, dis, b2):
    n, d = hs.shape
    blk = 1000
    grid = n // blk
    row_spec = pl.BlockSpec((blk, d), lambda i: (i, 0))
    return pl.pallas_call(
        _final_body,
        grid=(grid,),
        in_specs=[
            row_spec,
            row_spec,
            row_spec,
            pl.BlockSpec((blk, 1), lambda i: (i, 0)),
            pl.BlockSpec((1, d), lambda i: (0, 0)),
        ],
        out_specs=row_spec,
        out_shape=jax.ShapeDtypeStruct((n, d), jnp.float32),
    )(p0, p1, hs, dis, b2)


@jax.jit
def kernel(x, edge_index, W, b):
    n, d = x.shape
    e = edge_index.shape[1]
    assert e % (NW * ECH) == 0 and e % (NW * EC) == 0 and n <= N_PAD

    row_h = edge_index[0].reshape(NW, e // (NW * ECH), ECH)
    nchunk = e // (NW * EC)
    row_r = edge_index[0].reshape(NW, nchunk, EC)
    col_r = edge_index[1].reshape(NW, nchunk, EC)

    cnt = _degree_histogram(row_h)
    cnt_t = cnt[:, :n].T
    hs, dis = _prep(x, W, cnt_t)
    parts = _edge_scatter(hs, row_r, col_r)
    return _final(parts[0, :n], parts[1, :n], hs, dis, b.reshape(1, d))


# speedup vs baseline: 40.7736x; 1.0355x over previous
import functools

import jax
import jax.numpy as jnp
from jax import lax
from jax.experimental import pallas as pl
from jax.experimental.pallas import tpu as pltpu
from jax.experimental.pallas import tpu_sc as plsc

NC = 2
NS = 16
NW = NC * NS
L = 16

ECH = 100
EC = 125
NB = 2
N_PAD = 10240


def _hist_body(nchunk, row_hbm, cnt_hbm, row_v, ones_v, z_v, hist_sh, sem):
    del sem
    cid = lax.axis_index("c")
    sid = lax.axis_index("s")
    wid = cid * NS + sid
    rows_per_tile = N_PAD // NS

    pltpu.sync_copy(row_hbm.at[wid], row_v)

    def fill_ones(i, c):
        ones_v[pl.ds(i * L, L)] = jnp.full((L,), 1.0, jnp.float32)
        return c

    lax.fori_loop(0, ECH // L + 1, fill_ones, 0)

    def fill_zeros(i, c):
        z_v[pl.ds(i * L, L)] = jnp.zeros((L,), jnp.float32)
        return c

    lax.fori_loop(0, rows_per_tile // L, fill_zeros, 0)
    pltpu.sync_copy(z_v, hist_sh.at[pl.ds(sid * rows_per_tile, rows_per_tile)])
    plsc.subcore_barrier()

    def step(j, c):
        pltpu.sync_copy(ones_v.at[pl.ds(0, ECH)], hist_sh.at[row_v.at[j]], add=True)
        return c

    lax.fori_loop(0, nchunk, step, 0)
    plsc.subcore_barrier()
    pltpu.sync_copy(
        hist_sh.at[pl.ds(sid * rows_per_tile, rows_per_tile)],
        cnt_hbm.at[cid].at[pl.ds(sid * rows_per_tile, rows_per_tile)],
    )


def _degree_histogram(row_r):
    nchunk = row_r.shape[1]
    mesh = plsc.VectorSubcoreMesh(core_axis_name="c", subcore_axis_name="s")
    return pl.kernel(
        functools.partial(_hist_body, nchunk),
        out_type=jax.ShapeDtypeStruct((NC, N_PAD), jnp.float32),
        mesh=mesh,
        scratch_types=[
            pltpu.VMEM(row_r.shape[1:], jnp.int32),
            pltpu.VMEM(((ECH // L + 1) * L,), jnp.float32),
            pltpu.VMEM((N_PAD // NS,), jnp.float32),
            pltpu.VMEM_SHARED((N_PAD,), jnp.float32),
            pltpu.SemaphoreType.DMA,
        ],
    )(row_r)


def _prep_body(x_ref, w_ref, cnt_ref, hs_ref, dis_ref):
    cnt = cnt_ref[...]
    deg = cnt[:, 0:1] + cnt[:, 1:2] + 1.0
    dis = lax.rsqrt(deg)
    h = jnp.dot(x_ref[...], w_ref[...], preferred_element_type=jnp.float32)
    hs_ref[...] = h * dis
    dis_ref[...] = dis


def _prep(x, W, cnt_t):
    n, d = x.shape
    blk = 1000
    grid = n // blk
    return pl.pallas_call(
        _prep_body,
        grid=(grid,),
        in_specs=[
            pl.BlockSpec((blk, d), lambda i: (i, 0)),
            pl.BlockSpec((d, d), lambda i: (0, 0)),
            pl.BlockSpec((blk, 2), lambda i: (i, 0)),
        ],
        out_specs=[
            pl.BlockSpec((blk, d), lambda i: (i, 0)),
            pl.BlockSpec((blk, 1), lambda i: (i, 0)),
        ],
        out_shape=[
            jax.ShapeDtypeStruct((n, d), jnp.float32),
            jax.ShapeDtypeStruct((n, 1), jnp.float32),
        ],
    )(x, W, cnt_t)


def _scatter_body(nchunk, hs_hbm, row_hbm, col_hbm, part_hbm,
                  ri, col_v, bufs, acc_sh, *sems):
    isem = sems[:NB]
    gsem = sems[NB:]
    cid = lax.axis_index("c")
    sid = lax.axis_index("s")
    wid = cid * NS + sid
    rows_per_tile = N_PAD // NS
    zb = 128

    rsrc = row_hbm.at[wid]
    pltpu.sync_copy(col_hbm.at[wid], col_v)

    def zero_row(i, c):
        for k in range(bufs.shape[1] // L):
            bufs[i, pl.ds(k * L, L)] = jnp.zeros((L,), jnp.float32)
        return c

    lax.fori_loop(0, zb, zero_row, 0)
    for k in range(rows_per_tile // zb):
        pltpu.sync_copy(bufs.at[pl.ds(0, zb)],
                        acc_sh.at[pl.ds(sid * rows_per_tile + k * zb, zb)])
    plsc.subcore_barrier()

    def buf(i):
        return bufs.at[pl.ds(i * EC, EC)]

    def idx_load(j, i):
        pltpu.async_copy(rsrc.at[j], ri.at[i], isem[i])

    def idx_wait(j, i):
        pltpu.make_async_copy(rsrc.at[j], ri.at[i], isem[i]).wait()

    def gather(j, i):
        del j
        pltpu.async_copy(hs_hbm.at[ri.at[i]], buf(i), gsem[i])

    def gather_wait(j, i):
        del j
        pltpu.make_async_copy(hs_hbm.at[ri.at[i]], buf(i), gsem[i]).wait()

    def scatter(j, i):
        pltpu.sync_copy(buf(i), acc_sh.at[col_v.at[j]], add=True)

    idx_load(0, 0)
    idx_load(1, 1)
    idx_wait(0, 0)
    gather(0, 0)

    def body(k, c):
        for i in range(NB):
            j = NB * k + i
            i2 = (i + 1) % NB
            idx_wait(j + 1, i2)
            gather(j + 1, i2)
            gather_wait(j, i)
            scatter(j, i)
            idx_load(j + 2, i)
        return c

    lax.fori_loop(0, (nchunk - NB) // NB, body, 0)
    j = nchunk - NB
    for i0 in range(NB):
        i = (j + i0) % NB
        if i0 == 0:
            i2 = (i + 1) % NB
            idx_wait(j + 1, i2)
            gather(j + 1, i2)
        gather_wait(j + i0, i)
        scatter(j + i0, i)

    plsc.subcore_barrier()
    pltpu.sync_copy(
        acc_sh.at[pl.ds(sid * rows_per_tile, rows_per_tile)],
        part_hbm.at[cid].at[pl.ds(sid * rows_per_tile, rows_per_tile)],
    )


def _edge_scatter(hs, row_r, col_r):
    d = hs.shape[1]
    nchunk = row_r.shape[1]
    mesh = plsc.VectorSubcoreMesh(core_axis_name="c", subcore_axis_name="s")
    return pl.kernel(
        functools.partial(_scatter_body, nchunk),
        out_type=jax.ShapeDtypeStruct((NC, N_PAD, d), jnp.float32),
        mesh=mesh,
        scratch_types=[
            pltpu.VMEM((NB, EC), jnp.int32),
            pltpu.VMEM(col_r.shape[1:], jnp.int32),
            pltpu.VMEM((NB * EC, d), jnp.float32),
            pltpu.VMEM_SHARED((N_PAD, d), jnp.float32),
        ] + [pltpu.SemaphoreType.DMA] * (2 * NB),
    )(hs, row_r, col_r)


def _final_body(p_ref, hs_ref, dis_ref, b_ref, o_ref):
    p = p_ref[0] + p_ref[1]
    o_ref[...] = (p + hs_ref[...]) * dis_ref[...] + b_ref[...]


def _final(parts, hs, dis, b2):
    n, d = hs.shape
    blk = 1000
    grid = n // blk
    row_spec = pl.BlockSpec((blk, d), lambda i: (i, 0))
    return pl.pallas_call(
        _final_body,
        grid=(grid,),
        in_specs=[
            pl.BlockSpec((NC, blk, d), lambda i: (0, i, 0)),
            row_spec,
            pl.BlockSpec((blk, 1), lambda i: (i, 0)),
            pl.BlockSpec((1, d), lambda i: (0, 0)),
        ],
        out_specs=row_spec,
        out_shape=jax.ShapeDtypeStruct((n, d), jnp.float32),
    )(parts, hs, dis, b2)


@jax.jit
def kernel(x, edge_index, W, b):
    n, d = x.shape
    e = edge_index.shape[1]
    assert e % (NW * ECH) == 0 and e % (NW * EC) == 0 and n <= N_PAD

    row_h = edge_index[0].reshape(NW, e // (NW * ECH), ECH)
    nchunk = e // (NW * EC)
    row_r = edge_index[0].reshape(NW, nchunk, EC)
    col_r = edge_index[1].reshape(NW, nchunk, EC)

    cnt = _degree_histogram(row_h)
    cnt_t = cnt[:, :n].T
    hs, dis = _prep(x, W, cnt_t)
    parts = _edge_scatter(hs, row_r, col_r)
    return _final(parts, hs, dis, b.reshape(1, d))
